# Initial kernel scaffold; baseline (speedup 1.0000x reference)
import functools

import jax
import jax.numpy as jnp
from jax import lax
from jax.experimental import pallas as pl
from jax.experimental.pallas import tpu as pltpu
from jax.experimental.pallas import tpu_sc as plsc

N = 10000
E = 320000
BN_EPS = 1e-5

NC = 2
NS = 16
NW = NC * NS
EPW = E // NW
CHUNK = 125
NCH = EPW // CHUNK
RPT = N // NS
DEGW = 16

_MESH = plsc.VectorSubcoreMesh(core_axis_name="c", subcore_axis_name="s")


def _worker(core, sub):
    return sub * NC + core


def _deg_body(dst_hbm, ones_hbm, zeros_hbm, out_hbm, dst_v, ones_v, deg_sp,
              sem):
    c = lax.axis_index("c")
    s = lax.axis_index("s")
    wid = _worker(c, s)
    pltpu.sync_copy(zeros_hbm.at[pl.ds(s * RPT, RPT)],
                    deg_sp.at[pl.ds(s * RPT, RPT)])
    pltpu.sync_copy(ones_hbm, ones_v)
    pltpu.sync_copy(dst_hbm.at[pl.ds(wid * NCH, NCH)], dst_v)
    plsc.subcore_barrier()

    def body(i, carry):
        pltpu.sync_copy(ones_v, deg_sp.at[dst_v.at[i]], add=True)
        return carry

    lax.fori_loop(0, NCH, body, 0)
    plsc.subcore_barrier()
    pltpu.sync_copy(deg_sp.at[pl.ds(s * RPT, RPT)],
                    out_hbm.at[c, pl.ds(s * RPT, RPT)])


def _deg_kernel(dst2d, ones, zeros):
    return pl.kernel(
        _deg_body,
        out_type=jax.ShapeDtypeStruct((NC, N, DEGW), jnp.float32),
        mesh=_MESH,
        scratch_types=[
            pltpu.VMEM((NCH, CHUNK), jnp.int32),
            pltpu.VMEM((CHUNK, DEGW), jnp.float32),
            pltpu.VMEM_SHARED((N, DEGW), jnp.float32),
            pltpu.SemaphoreType.DMA,
        ],
    )(dst2d, ones, zeros)


def _agg_body(y_hbm, src_hbm, dst_hbm, zeros_hbm, out_hbm, src_v, dst_v,
              rows_a, rows_b, acc_sp, sem_a, sem_b):
    c = lax.axis_index("c")
    s = lax.axis_index("s")
    wid = _worker(c, s)
    pltpu.sync_copy(zeros_hbm.at[pl.ds(s * RPT, RPT)],
                    acc_sp.at[pl.ds(s * RPT, RPT)])
    pltpu.sync_copy(src_hbm.at[pl.ds(wid * NCH, NCH)], src_v)
    pltpu.sync_copy(dst_hbm.at[pl.ds(wid * NCH, NCH)], dst_v)
    plsc.subcore_barrier()

    pltpu.async_copy(y_hbm.at[src_v.at[0]], rows_a, sem_a)
    pltpu.async_copy(y_hbm.at[src_v.at[1]], rows_b, sem_b)

    def body(j, carry):
        i = 2 * j
        pltpu.make_async_copy(y_hbm.at[src_v.at[0]], rows_a, sem_a).wait()
        pltpu.sync_copy(rows_a, acc_sp.at[dst_v.at[i]], add=True)

        @pl.when(i + 2 < NCH)
        def _():
            pltpu.async_copy(y_hbm.at[src_v.at[i + 2]], rows_a, sem_a)

        pltpu.make_async_copy(y_hbm.at[src_v.at[0]], rows_b, sem_b).wait()
        pltpu.sync_copy(rows_b, acc_sp.at[dst_v.at[i + 1]], add=True)

        @pl.when(i + 3 < NCH)
        def _():
            pltpu.async_copy(y_hbm.at[src_v.at[i + 3]], rows_b, sem_b)

        return carry

    lax.fori_loop(0, NCH // 2, body, 0)
    plsc.subcore_barrier()
    pltpu.sync_copy(acc_sp.at[pl.ds(s * RPT, RPT)],
                    out_hbm.at[c, pl.ds(s * RPT, RPT)])


def _agg_kernel(y, src2d, dst2d, zeros, d):
    return pl.kernel(
        _agg_body,
        out_type=jax.ShapeDtypeStruct((NC, N, d), jnp.float32),
        mesh=_MESH,
        scratch_types=[
            pltpu.VMEM((NCH, CHUNK), jnp.int32),
            pltpu.VMEM((NCH, CHUNK), jnp.int32),
            pltpu.VMEM((CHUNK, d), jnp.float32),
            pltpu.VMEM((CHUNK, d), jnp.float32),
            pltpu.VMEM_SHARED((N, d), jnp.float32),
            pltpu.SemaphoreType.DMA,
            pltpu.SemaphoreType.DMA,
        ],
    )(y, src2d, dst2d, zeros)


def _tc1_body(x_ref, w1_ref, degp_ref, y1_ref, dis_ref):
    deg = degp_ref[0, :, 0:1] + degp_ref[1, :, 0:1] + 1.0
    dis = lax.rsqrt(deg)
    xw = lax.dot_general(x_ref[...], w1_ref[...], (((1,), (1,)), ((), ())),
                         preferred_element_type=jnp.float32)
    y1_ref[...] = xw * dis
    dis_ref[...] = dis


def _tc2_body(aggp_ref, y1_ref, dis_ref, w2_ref, b1_ref, g1s_ref, be1_ref,
              y2_ref):
    dis = dis_ref[...]
    a = (aggp_ref[0] + aggp_ref[1] + y1_ref[...]) * dis + b1_ref[...]
    h1 = jnp.maximum(a * g1s_ref[...] + be1_ref[...], 0.0)
    y2_ref[...] = lax.dot_general(h1, w2_ref[...], (((1,), (1,)), ((), ())),
                                  preferred_element_type=jnp.float32) * dis


def _tc3_body(aggp_ref, y2_ref, dis_ref, fcw_ref, b2_ref, g2s_ref, be2_ref,
              fcb_ref, out_ref):
    a = (aggp_ref[0] + aggp_ref[1] + y2_ref[...]) * dis_ref[...] + b2_ref[...]
    h2 = jnp.maximum(a * g2s_ref[...] + be2_ref[...], 0.0)
    out_ref[...] = lax.dot_general(h2, fcw_ref[...], (((1,), (1,)), ((), ())),
                                   preferred_element_type=jnp.float32) \
        + fcb_ref[...]


def _tc_call(body, out_shapes):
    return pl.pallas_call(body, out_shape=out_shapes)


@jax.jit
def _run(x, edge_index, W1, b1, g1, be1, W2, b2, g2, be2, fcW, fcb):
    src2d = edge_index[0].reshape(E // CHUNK, CHUNK)
    dst2d = edge_index[1].reshape(E // CHUNK, CHUNK)
    ones = jnp.ones((CHUNK, DEGW), jnp.float32)
    z16 = jnp.zeros((N, DEGW), jnp.float32)
    z64 = jnp.zeros((N, 64), jnp.float32)
    z32 = jnp.zeros((N, 32), jnp.float32)
    bn_s = 1.0 / jnp.sqrt(1.0 + BN_EPS)
    g1s = (g1 * bn_s).reshape(1, 64)
    g2s = (g2 * bn_s).reshape(1, 32)

    degp = _deg_kernel(dst2d, ones, z16)

    y1, dis = _tc_call(_tc1_body, (
        jax.ShapeDtypeStruct((N, 64), jnp.float32),
        jax.ShapeDtypeStruct((N, 1), jnp.float32),
    ))(x, W1, degp)

    agg1p = _agg_kernel(y1, src2d, dst2d, z64, 64)

    y2 = _tc_call(_tc2_body, jax.ShapeDtypeStruct((N, 32), jnp.float32))(
        agg1p, y1, dis, W2, b1.reshape(1, 64), g1s, be1.reshape(1, 64))

    agg2p = _agg_kernel(y2, src2d, dst2d, z32, 32)

    out = _tc_call(_tc3_body, jax.ShapeDtypeStruct((N, 1), jnp.float32))(
        agg2p, y2, dis, fcW, b2.reshape(1, 32), g2s, be2.reshape(1, 32),
        fcb.reshape(1, 1))
    return out


def kernel(x, edge_index, W1, b1, g1, be1, W2, b2, g2, be2, fcW, fcb):
    return _run(x, edge_index, W1, b1, g1, be1, W2, b2, g2, be2, fcW, fcb)


# speedup vs baseline: 39.9099x; 39.9099x over previous
import functools

import jax
import jax.numpy as jnp
from jax import lax
from jax.experimental import pallas as pl
from jax.experimental.pallas import tpu as pltpu
from jax.experimental.pallas import tpu_sc as plsc

N = 10000
E = 320000
BN_EPS = 1e-5

NC = 2
NS = 16
NW = NC * NS
EPW = E // NW
CHUNK = 125
NCH = EPW // CHUNK
RPT = 624
TAIL = N - NS * RPT
TAIL_OFF = NS * RPT
DEGW = 16

_MESH = plsc.VectorSubcoreMesh(core_axis_name="c", subcore_axis_name="s")


def _worker(core, sub):
    return sub * NC + core


def _each_tile_slab(s, fn):
    fn(pl.multiple_of(s * RPT, 8), RPT)

    @pl.when(s == NS - 1)
    def _():
        fn(TAIL_OFF, TAIL)


def _deg_body(dst_hbm, ones_hbm, zeros_hbm, out_hbm, dst_v, ones_v, deg_sp,
              sem):
    c = lax.axis_index("c")
    s = lax.axis_index("s")
    wid = _worker(c, s)
    _each_tile_slab(s, lambda o, n: pltpu.sync_copy(
        zeros_hbm.at[pl.ds(o, n)], deg_sp.at[pl.ds(o, n)]))
    pltpu.sync_copy(ones_hbm, ones_v)
    pltpu.sync_copy(dst_hbm.at[pl.ds(wid * NCH, NCH)], dst_v)
    plsc.subcore_barrier()

    def body(i, carry):
        pltpu.sync_copy(ones_v, deg_sp.at[dst_v.at[i]], add=True)
        return carry

    lax.fori_loop(0, NCH, body, 0)
    plsc.subcore_barrier()
    _each_tile_slab(s, lambda o, n: pltpu.sync_copy(
        deg_sp.at[pl.ds(o, n)], out_hbm.at[c, pl.ds(o, n)]))


def _deg_kernel(dst2d, ones, zeros):
    return pl.kernel(
        _deg_body,
        out_type=jax.ShapeDtypeStruct((NC, N, DEGW), jnp.float32),
        mesh=_MESH,
        compiler_params=pltpu.CompilerParams(use_tc_tiling_on_sc=False),
        scratch_types=[
            pltpu.VMEM((NCH, CHUNK), jnp.int32),
            pltpu.VMEM((CHUNK, DEGW), jnp.float32),
            pltpu.VMEM_SHARED((N, DEGW), jnp.float32),
            pltpu.SemaphoreType.DMA,
        ],
    )(dst2d, ones, zeros)


def _agg_body(y_hbm, src_hbm, dst_hbm, zeros_hbm, out_hbm, src_v, dst_v,
              rows_a, rows_b, acc_sp, sem_a, sem_b):
    c = lax.axis_index("c")
    s = lax.axis_index("s")
    wid = _worker(c, s)
    _each_tile_slab(s, lambda o, n: pltpu.sync_copy(
        zeros_hbm.at[pl.ds(o, n)], acc_sp.at[pl.ds(o, n)]))
    pltpu.sync_copy(src_hbm.at[pl.ds(wid * NCH, NCH)], src_v)
    pltpu.sync_copy(dst_hbm.at[pl.ds(wid * NCH, NCH)], dst_v)
    plsc.subcore_barrier()

    pltpu.async_copy(y_hbm.at[src_v.at[0]], rows_a, sem_a)
    pltpu.async_copy(y_hbm.at[src_v.at[1]], rows_b, sem_b)

    def body(j, carry):
        i = 2 * j
        pltpu.make_async_copy(y_hbm.at[src_v.at[0]], rows_a, sem_a).wait()
        pltpu.sync_copy(rows_a, acc_sp.at[dst_v.at[i]], add=True)

        @pl.when(i + 2 < NCH)
        def _():
            pltpu.async_copy(y_hbm.at[src_v.at[i + 2]], rows_a, sem_a)

        pltpu.make_async_copy(y_hbm.at[src_v.at[0]], rows_b, sem_b).wait()
        pltpu.sync_copy(rows_b, acc_sp.at[dst_v.at[i + 1]], add=True)

        @pl.when(i + 3 < NCH)
        def _():
            pltpu.async_copy(y_hbm.at[src_v.at[i + 3]], rows_b, sem_b)

        return carry

    lax.fori_loop(0, NCH // 2, body, 0)
    plsc.subcore_barrier()
    _each_tile_slab(s, lambda o, n: pltpu.sync_copy(
        acc_sp.at[pl.ds(o, n)], out_hbm.at[c, pl.ds(o, n)]))


def _agg_kernel(y, src2d, dst2d, zeros, d):
    return pl.kernel(
        _agg_body,
        out_type=jax.ShapeDtypeStruct((NC, N, d), jnp.float32),
        mesh=_MESH,
        compiler_params=pltpu.CompilerParams(use_tc_tiling_on_sc=False),
        scratch_types=[
            pltpu.VMEM((NCH, CHUNK), jnp.int32),
            pltpu.VMEM((NCH, CHUNK), jnp.int32),
            pltpu.VMEM((CHUNK, d), jnp.float32),
            pltpu.VMEM((CHUNK, d), jnp.float32),
            pltpu.VMEM_SHARED((N, d), jnp.float32),
            pltpu.SemaphoreType.DMA,
            pltpu.SemaphoreType.DMA,
        ],
    )(y, src2d, dst2d, zeros)


def _tc1_body(x_ref, w1_ref, degp_ref, y1_ref, dis_ref):
    deg = degp_ref[0, :, 0:1] + degp_ref[1, :, 0:1] + 1.0
    dis = lax.rsqrt(deg)
    xw = lax.dot_general(x_ref[...], w1_ref[...], (((1,), (1,)), ((), ())),
                         preferred_element_type=jnp.float32)
    y1_ref[...] = xw * dis
    dis_ref[...] = dis


def _tc2_body(aggp_ref, y1_ref, dis_ref, w2_ref, b1_ref, g1s_ref, be1_ref,
              y2_ref):
    dis = dis_ref[...]
    a = (aggp_ref[0] + aggp_ref[1] + y1_ref[...]) * dis + b1_ref[...]
    h1 = jnp.maximum(a * g1s_ref[...] + be1_ref[...], 0.0)
    y2_ref[...] = lax.dot_general(h1, w2_ref[...], (((1,), (1,)), ((), ())),
                                  preferred_element_type=jnp.float32) * dis


def _tc3_body(aggp_ref, y2_ref, dis_ref, fcw_ref, b2_ref, g2s_ref, be2_ref,
              fcb_ref, out_ref):
    a = (aggp_ref[0] + aggp_ref[1] + y2_ref[...]) * dis_ref[...] + b2_ref[...]
    h2 = jnp.maximum(a * g2s_ref[...] + be2_ref[...], 0.0)
    out_ref[...] = jnp.sum(h2 * fcw_ref[...], axis=1, keepdims=True) \
        + fcb_ref[...]


def _tc_call(body, out_shapes):
    return pl.pallas_call(body, out_shape=out_shapes)


@jax.jit
def _run(x, edge_index, W1, b1, g1, be1, W2, b2, g2, be2, fcW, fcb):
    src2d = edge_index[0].reshape(E // CHUNK, CHUNK)
    dst2d = edge_index[1].reshape(E // CHUNK, CHUNK)
    ones = jnp.ones((CHUNK, DEGW), jnp.float32)
    z16 = jnp.zeros((N, DEGW), jnp.float32)
    z64 = jnp.zeros((N, 64), jnp.float32)
    z32 = jnp.zeros((N, 32), jnp.float32)
    bn_s = 1.0 / jnp.sqrt(1.0 + BN_EPS)
    g1s = (g1 * bn_s).reshape(1, 64)
    g2s = (g2 * bn_s).reshape(1, 32)

    degp = _deg_kernel(dst2d, ones, z16)

    y1, dis = _tc_call(_tc1_body, (
        jax.ShapeDtypeStruct((N, 64), jnp.float32),
        jax.ShapeDtypeStruct((N, 1), jnp.float32),
    ))(x, W1, degp)

    agg1p = _agg_kernel(y1, src2d, dst2d, z64, 64)

    y2 = _tc_call(_tc2_body, jax.ShapeDtypeStruct((N, 32), jnp.float32))(
        agg1p, y1, dis, W2, b1.reshape(1, 64), g1s, be1.reshape(1, 64))

    agg2p = _agg_kernel(y2, src2d, dst2d, z32, 32)

    out = _tc_call(_tc3_body, jax.ShapeDtypeStruct((N, 1), jnp.float32))(
        agg2p, y2, dis, fcW, b2.reshape(1, 32), g2s, be2.reshape(1, 32),
        fcb.reshape(1, 1))
    return out


def kernel(x, edge_index, W1, b1, g1, be1, W2, b2, g2, be2, fcW, fcb):
    return _run(x, edge_index, W1, b1, g1, be1, W2, b2, g2, be2, fcW, fcb)


# speedup vs baseline: 44.0539x; 1.1038x over previous
import functools

import jax
import jax.numpy as jnp
from jax import lax
from jax.experimental import pallas as pl
from jax.experimental.pallas import tpu as pltpu
from jax.experimental.pallas import tpu_sc as plsc

N = 10000
E = 320000
BN_EPS = 1e-5

NC = 2
NS = 16
NW = NC * NS
EPW = E // NW
CHUNK = 125
NCH = EPW // CHUNK
RPT = 624
TAIL = N - NS * RPT
TAIL_OFF = NS * RPT
DEGW = 8
NBUF = 4

_MESH = plsc.VectorSubcoreMesh(core_axis_name="c", subcore_axis_name="s")


def _worker(core, sub):
    return sub * NC + core


def _each_tile_slab(s, fn):
    fn(pl.multiple_of(s * RPT, 8), RPT)

    @pl.when(s == NS - 1)
    def _():
        fn(TAIL_OFF, TAIL)


def _deg_body(dst_hbm, ones_hbm, zeros_hbm, out_hbm, dst_v, ones_v, deg_sp,
              sem):
    c = lax.axis_index("c")
    s = lax.axis_index("s")
    wid = _worker(c, s)
    _each_tile_slab(s, lambda o, n: pltpu.sync_copy(
        zeros_hbm.at[pl.ds(o, n)], deg_sp.at[pl.ds(o, n)]))
    pltpu.sync_copy(ones_hbm, ones_v)
    pltpu.sync_copy(dst_hbm.at[pl.ds(wid * NCH, NCH)], dst_v)
    plsc.subcore_barrier()

    def body(i, carry):
        pltpu.sync_copy(ones_v, deg_sp.at[dst_v.at[i]], add=True)
        return carry

    lax.fori_loop(0, NCH, body, 0)
    plsc.subcore_barrier()
    _each_tile_slab(s, lambda o, n: pltpu.sync_copy(
        deg_sp.at[pl.ds(o, n)], out_hbm.at[c, pl.ds(o, n)]))


def _deg_kernel(dst2d, ones, zeros):
    return pl.kernel(
        _deg_body,
        out_type=jax.ShapeDtypeStruct((NC, N, DEGW), jnp.float32),
        mesh=_MESH,
        compiler_params=pltpu.CompilerParams(use_tc_tiling_on_sc=False),
        scratch_types=[
            pltpu.VMEM((NCH, CHUNK), jnp.int32),
            pltpu.VMEM((CHUNK, DEGW), jnp.float32),
            pltpu.VMEM_SHARED((N, DEGW), jnp.float32),
            pltpu.SemaphoreType.DMA,
        ],
    )(dst2d, ones, zeros)


def _agg_body(y_hbm, src_hbm, dst_hbm, zeros_hbm, out_hbm, src_v, dst_v,
              *bufs_and_sems):
    bufs = bufs_and_sems[:NBUF]
    acc_sp = bufs_and_sems[NBUF]
    gsems = bufs_and_sems[NBUF + 1:2 * NBUF + 1]
    ssems = bufs_and_sems[2 * NBUF + 1:]
    c = lax.axis_index("c")
    s = lax.axis_index("s")
    wid = _worker(c, s)
    _each_tile_slab(s, lambda o, n: pltpu.sync_copy(
        zeros_hbm.at[pl.ds(o, n)], acc_sp.at[pl.ds(o, n)]))
    pltpu.sync_copy(src_hbm.at[pl.ds(wid * NCH, NCH)], src_v)
    pltpu.sync_copy(dst_hbm.at[pl.ds(wid * NCH, NCH)], dst_v)
    plsc.subcore_barrier()

    for k in range(NBUF):
        pltpu.async_copy(y_hbm.at[src_v.at[k]], bufs[k], gsems[k])

    def body(j, carry):
        i = NBUF * j
        for k in range(NBUF):
            pltpu.make_async_copy(y_hbm.at[src_v.at[0]], bufs[k],
                                  gsems[k]).wait()
            pltpu.async_copy(bufs[k], acc_sp.at[dst_v.at[i + k]], ssems[k],
                             add=True)
        for k in range(NBUF):
            @pl.when(i + NBUF + k < NCH)
            def _(k=k):
                pltpu.make_async_copy(bufs[k], acc_sp.at[dst_v.at[0]],
                                      ssems[k]).wait()
                pltpu.async_copy(y_hbm.at[src_v.at[i + NBUF + k]], bufs[k],
                                 gsems[k])
        return carry

    lax.fori_loop(0, NCH // NBUF, body, 0)
    for k in range(NBUF):
        pltpu.make_async_copy(bufs[k], acc_sp.at[dst_v.at[0]],
                              ssems[k]).wait()
    plsc.subcore_barrier()
    _each_tile_slab(s, lambda o, n: pltpu.sync_copy(
        acc_sp.at[pl.ds(o, n)], out_hbm.at[c, pl.ds(o, n)]))


def _agg_kernel(y, src2d, dst2d, zeros, d):
    return pl.kernel(
        _agg_body,
        out_type=jax.ShapeDtypeStruct((NC, N, d), jnp.float32),
        mesh=_MESH,
        compiler_params=pltpu.CompilerParams(use_tc_tiling_on_sc=False),
        scratch_types=[
            pltpu.VMEM((NCH, CHUNK), jnp.int32),
            pltpu.VMEM((NCH, CHUNK), jnp.int32),
        ] + [pltpu.VMEM((CHUNK, d), jnp.float32) for _ in range(NBUF)] + [
            pltpu.VMEM_SHARED((N, d), jnp.float32),
        ] + [pltpu.SemaphoreType.DMA for _ in range(2 * NBUF)],
    )(y, src2d, dst2d, zeros)


def _tc1_body(x_ref, w1_ref, degp_ref, y1_ref, dis_ref):
    deg = degp_ref[0, :, 0:1] + degp_ref[1, :, 0:1] + 1.0
    dis = lax.rsqrt(deg)
    xw = lax.dot_general(x_ref[...], w1_ref[...], (((1,), (1,)), ((), ())),
                         preferred_element_type=jnp.float32)
    y1_ref[...] = xw * dis
    dis_ref[...] = dis


def _tc2_body(aggp_ref, y1_ref, dis_ref, w2_ref, b1_ref, g1s_ref, be1_ref,
              y2_ref):
    dis = dis_ref[...]
    a = (aggp_ref[0] + aggp_ref[1] + y1_ref[...]) * dis + b1_ref[...]
    h1 = jnp.maximum(a * g1s_ref[...] + be1_ref[...], 0.0)
    y2_ref[...] = lax.dot_general(h1, w2_ref[...], (((1,), (1,)), ((), ())),
                                  preferred_element_type=jnp.float32) * dis


def _tc3_body(aggp_ref, y2_ref, dis_ref, fcw_ref, b2_ref, g2s_ref, be2_ref,
              fcb_ref, out_ref):
    a = (aggp_ref[0] + aggp_ref[1] + y2_ref[...]) * dis_ref[...] + b2_ref[...]
    h2 = jnp.maximum(a * g2s_ref[...] + be2_ref[...], 0.0)
    out_ref[...] = jnp.sum(h2 * fcw_ref[...], axis=1, keepdims=True) \
        + fcb_ref[...]


def _tc_call(body, out_shapes):
    return pl.pallas_call(body, out_shape=out_shapes)


@jax.jit
def _run(x, edge_index, W1, b1, g1, be1, W2, b2, g2, be2, fcW, fcb):
    src2d = edge_index[0].reshape(E // CHUNK, CHUNK)
    dst2d = edge_index[1].reshape(E // CHUNK, CHUNK)
    ones = jnp.ones((CHUNK, DEGW), jnp.float32)
    z16 = jnp.zeros((N, DEGW), jnp.float32)
    z64 = jnp.zeros((N, 64), jnp.float32)
    z32 = jnp.zeros((N, 32), jnp.float32)
    bn_s = 1.0 / jnp.sqrt(1.0 + BN_EPS)
    g1s = (g1 * bn_s).reshape(1, 64)
    g2s = (g2 * bn_s).reshape(1, 32)

    degp = _deg_kernel(dst2d, ones, z16)

    y1, dis = _tc_call(_tc1_body, (
        jax.ShapeDtypeStruct((N, 64), jnp.float32),
        jax.ShapeDtypeStruct((N, 1), jnp.float32),
    ))(x, W1, degp)

    agg1p = _agg_kernel(y1, src2d, dst2d, z64, 64)

    y2 = _tc_call(_tc2_body, jax.ShapeDtypeStruct((N, 32), jnp.float32))(
        agg1p, y1, dis, W2, b1.reshape(1, 64), g1s, be1.reshape(1, 64))

    agg2p = _agg_kernel(y2, src2d, dst2d, z32, 32)

    out = _tc_call(_tc3_body, jax.ShapeDtypeStruct((N, 1), jnp.float32))(
        agg2p, y2, dis, fcW, b2.reshape(1, 32), g2s, be2.reshape(1, 32),
        fcb.reshape(1, 1))
    return out


def kernel(x, edge_index, W1, b1, g1, be1, W2, b2, g2, be2, fcW, fcb):
    return _run(x, edge_index, W1, b1, g1, be1, W2, b2, g2, be2, fcW, fcb)


# speedup vs baseline: 46.2184x; 1.0491x over previous
import functools

import jax
import jax.numpy as jnp
from jax import lax
from jax.experimental import pallas as pl
from jax.experimental.pallas import tpu as pltpu
from jax.experimental.pallas import tpu_sc as plsc

N = 10000
E = 320000
BN_EPS = 1e-5

NC = 2
NS = 16
NW = NC * NS
EPW = E // NW
CHUNK = 125
NCH = EPW // CHUNK
RPT = 624
TAIL = N - NS * RPT
TAIL_OFF = NS * RPT
DEGW = 8
NBUF = 4

_MESH = plsc.VectorSubcoreMesh(core_axis_name="c", subcore_axis_name="s")


def _worker(core, sub):
    return sub * NC + core


def _each_tile_slab(s, fn):
    fn(pl.multiple_of(s * RPT, 8), RPT)

    @pl.when(s == NS - 1)
    def _():
        fn(TAIL_OFF, TAIL)


def _deg_body(ei_hbm, ones_hbm, zeros_hbm, out_hbm, dst_v, ones_v, deg_sp,
              sem):
    c = lax.axis_index("c")
    s = lax.axis_index("s")
    wid = _worker(c, s)
    _each_tile_slab(s, lambda o, n: pltpu.sync_copy(
        zeros_hbm.at[pl.ds(o, n)], deg_sp.at[pl.ds(o, n)]))
    pltpu.sync_copy(ones_hbm, ones_v)
    pltpu.sync_copy(ei_hbm.at[1, pl.ds(wid * NCH, NCH)], dst_v)
    plsc.subcore_barrier()

    def body(i, carry):
        pltpu.sync_copy(ones_v, deg_sp.at[dst_v.at[i]], add=True)
        return carry

    lax.fori_loop(0, NCH, body, 0)
    plsc.subcore_barrier()
    _each_tile_slab(s, lambda o, n: pltpu.sync_copy(
        deg_sp.at[pl.ds(o, n)], out_hbm.at[c, pl.ds(o, n)]))


def _deg_kernel(ei3, ones, zeros):
    return pl.kernel(
        _deg_body,
        out_type=jax.ShapeDtypeStruct((NC, N, DEGW), jnp.float32),
        mesh=_MESH,
        compiler_params=pltpu.CompilerParams(use_tc_tiling_on_sc=False),
        scratch_types=[
            pltpu.VMEM((NCH, CHUNK), jnp.int32),
            pltpu.VMEM((CHUNK, DEGW), jnp.float32),
            pltpu.VMEM_SHARED((N, DEGW), jnp.float32),
            pltpu.SemaphoreType.DMA,
        ],
    )(ei3, ones, zeros)


def _agg_body(y_hbm, ei_hbm, zeros_hbm, out_hbm, src_v, dst_v,
              *bufs_and_sems):
    bufs = bufs_and_sems[:NBUF]
    acc_sp = bufs_and_sems[NBUF]
    gsems = bufs_and_sems[NBUF + 1:2 * NBUF + 1]
    ssems = bufs_and_sems[2 * NBUF + 1:]
    c = lax.axis_index("c")
    s = lax.axis_index("s")
    wid = _worker(c, s)
    @pl.when(c == 0)
    def _():
        _each_tile_slab(s, lambda o, n: pltpu.sync_copy(
            y_hbm.at[pl.ds(o, n)], acc_sp.at[pl.ds(o, n)]))

    @pl.when(c != 0)
    def _():
        _each_tile_slab(s, lambda o, n: pltpu.sync_copy(
            zeros_hbm.at[pl.ds(o, n)], acc_sp.at[pl.ds(o, n)]))

    pltpu.sync_copy(ei_hbm.at[0, pl.ds(wid * NCH, NCH)], src_v)
    pltpu.sync_copy(ei_hbm.at[1, pl.ds(wid * NCH, NCH)], dst_v)
    plsc.subcore_barrier()

    for k in range(NBUF):
        pltpu.async_copy(y_hbm.at[src_v.at[k]], bufs[k], gsems[k])

    def body(j, carry):
        i = NBUF * j
        for k in range(NBUF):
            pltpu.make_async_copy(y_hbm.at[src_v.at[0]], bufs[k],
                                  gsems[k]).wait()
            pltpu.async_copy(bufs[k], acc_sp.at[dst_v.at[i + k]], ssems[k],
                             add=True)
        for k in range(NBUF):
            @pl.when(i + NBUF + k < NCH)
            def _(k=k):
                pltpu.make_async_copy(bufs[k], acc_sp.at[dst_v.at[0]],
                                      ssems[k]).wait()
                pltpu.async_copy(y_hbm.at[src_v.at[i + NBUF + k]], bufs[k],
                                 gsems[k])
        return carry

    lax.fori_loop(0, NCH // NBUF, body, 0)
    for k in range(NBUF):
        pltpu.make_async_copy(bufs[k], acc_sp.at[dst_v.at[0]],
                              ssems[k]).wait()
    plsc.subcore_barrier()
    _each_tile_slab(s, lambda o, n: pltpu.sync_copy(
        acc_sp.at[pl.ds(o, n)], out_hbm.at[c, pl.ds(o, n)]))


def _agg_kernel(y, ei3, zeros, d):
    return pl.kernel(
        _agg_body,
        out_type=jax.ShapeDtypeStruct((NC, N, d), jnp.float32),
        mesh=_MESH,
        compiler_params=pltpu.CompilerParams(use_tc_tiling_on_sc=False),
        scratch_types=[
            pltpu.VMEM((NCH, CHUNK), jnp.int32),
            pltpu.VMEM((NCH, CHUNK), jnp.int32),
        ] + [pltpu.VMEM((CHUNK, d), jnp.float32) for _ in range(NBUF)] + [
            pltpu.VMEM_SHARED((N, d), jnp.float32),
        ] + [pltpu.SemaphoreType.DMA for _ in range(2 * NBUF)],
    )(y, ei3, zeros)


def _xw_body(x_ref, w1_ref, xw_ref):
    xw_ref[...] = lax.dot_general(x_ref[...], w1_ref[...],
                                  (((1,), (1,)), ((), ())),
                                  preferred_element_type=jnp.float32)


def _scale_body(xw_ref, degp_ref, y1_ref, dis_ref):
    deg = degp_ref[0, :, 0:1] + degp_ref[1, :, 0:1] + 1.0
    dis = lax.rsqrt(deg)
    y1_ref[...] = xw_ref[...] * dis
    dis_ref[...] = dis


def _tc2_body(aggp_ref, dis_ref, w2_ref, b1_ref, g1s_ref, be1_ref,
              y2_ref):
    dis = dis_ref[...]
    a = (aggp_ref[0] + aggp_ref[1]) * dis + b1_ref[...]
    h1 = jnp.maximum(a * g1s_ref[...] + be1_ref[...], 0.0)
    y2_ref[...] = lax.dot_general(h1, w2_ref[...], (((1,), (1,)), ((), ())),
                                  preferred_element_type=jnp.float32) * dis


def _tc3_body(aggp_ref, dis_ref, fcw_ref, b2_ref, g2s_ref, be2_ref,
              fcb_ref, out_ref):
    a = (aggp_ref[0] + aggp_ref[1]) * dis_ref[...] + b2_ref[...]
    h2 = jnp.maximum(a * g2s_ref[...] + be2_ref[...], 0.0)
    out_ref[...] = jnp.sum(h2 * fcw_ref[...], axis=1, keepdims=True) \
        + fcb_ref[...]


def _tc_call(body, out_shapes):
    return pl.pallas_call(body, out_shape=out_shapes)


@jax.jit
def _run(x, edge_index, W1, b1, g1, be1, W2, b2, g2, be2, fcW, fcb):
    ei3 = edge_index.reshape(2, E // CHUNK, CHUNK)
    ones = jnp.ones((CHUNK, DEGW), jnp.float32)
    z8 = jnp.zeros((N, DEGW), jnp.float32)
    z64 = jnp.zeros((N, 64), jnp.float32)
    z32 = jnp.zeros((N, 32), jnp.float32)
    bn_s = 1.0 / jnp.sqrt(1.0 + BN_EPS)
    g1s = (g1 * bn_s).reshape(1, 64)
    g2s = (g2 * bn_s).reshape(1, 32)

    xw = _tc_call(_xw_body, jax.ShapeDtypeStruct((N, 64), jnp.float32))(x, W1)
    degp = _deg_kernel(ei3, ones, z8)

    y1, dis = _tc_call(_scale_body, (
        jax.ShapeDtypeStruct((N, 64), jnp.float32),
        jax.ShapeDtypeStruct((N, 1), jnp.float32),
    ))(xw, degp)

    agg1p = _agg_kernel(y1, ei3, z64, 64)

    y2 = _tc_call(_tc2_body, jax.ShapeDtypeStruct((N, 32), jnp.float32))(
        agg1p, dis, W2, b1.reshape(1, 64), g1s, be1.reshape(1, 64))

    agg2p = _agg_kernel(y2, ei3, z32, 32)

    out = _tc_call(_tc3_body, jax.ShapeDtypeStruct((N, 1), jnp.float32))(
        agg2p, dis, fcW, b2.reshape(1, 32), g2s, be2.reshape(1, 32),
        fcb.reshape(1, 1))
    return out


def kernel(x, edge_index, W1, b1, g1, be1, W2, b2, g2, be2, fcW, fcb):
    return _run(x, edge_index, W1, b1, g1, be1, W2, b2, g2, be2, fcW, fcb)


# speedup vs baseline: 49.7444x; 1.0763x over previous
import functools

import jax
import jax.numpy as jnp
from jax import lax
from jax.experimental import pallas as pl
from jax.experimental.pallas import tpu as pltpu
from jax.experimental.pallas import tpu_sc as plsc

N = 10000
E = 320000
BN_EPS = 1e-5

NC = 2
NS = 16
NW = NC * NS
EPW = E // NW
CHUNK = 125
NCH = EPW // CHUNK
RPT = 624
TAIL = N - NS * RPT
TAIL_OFF = NS * RPT
DEGW = 8
NBUF = 4

_MESH = plsc.VectorSubcoreMesh(core_axis_name="c", subcore_axis_name="s")


def _worker(core, sub):
    return sub * NC + core


def _each_tile_slab(s, fn):
    fn(pl.multiple_of(s * RPT, 8), RPT)

    @pl.when(s == NS - 1)
    def _():
        fn(TAIL_OFF, TAIL)


def _deg_body(ei_hbm, ones_hbm, zeros_hbm, out_hbm, dst_v, ones_v, deg_sp,
              sem):
    c = lax.axis_index("c")
    s = lax.axis_index("s")
    wid = _worker(c, s)
    _each_tile_slab(s, lambda o, n: pltpu.sync_copy(
        zeros_hbm.at[pl.ds(o, n)], deg_sp.at[pl.ds(o, n)]))
    pltpu.sync_copy(ones_hbm, ones_v)
    pltpu.sync_copy(ei_hbm.at[1, pl.ds(wid * NCH, NCH)], dst_v)
    plsc.subcore_barrier()

    def body(i, carry):
        pltpu.sync_copy(ones_v, deg_sp.at[dst_v.at[i]], add=True)
        return carry

    lax.fori_loop(0, NCH, body, 0)
    plsc.subcore_barrier()
    _each_tile_slab(s, lambda o, n: pltpu.sync_copy(
        deg_sp.at[pl.ds(o, n)], out_hbm.at[c, pl.ds(o, n)]))


def _deg_kernel(ei3, ones, zeros):
    return pl.kernel(
        _deg_body,
        out_type=jax.ShapeDtypeStruct((NC, N, DEGW), jnp.float32),
        mesh=_MESH,
        compiler_params=pltpu.CompilerParams(use_tc_tiling_on_sc=False),
        scratch_types=[
            pltpu.VMEM((NCH, CHUNK), jnp.int32),
            pltpu.VMEM((CHUNK, DEGW), jnp.float32),
            pltpu.VMEM_SHARED((N, DEGW), jnp.float32),
            pltpu.SemaphoreType.DMA,
        ],
    )(ei3, ones, zeros)


def _agg_body(d, y_hbm, ei_hbm, zeros_hbm, out_hbm, src_v, dst_v,
              *bufs_and_sems):
    bufs = bufs_and_sems[:NBUF]
    acc_sp = bufs_and_sems[NBUF]
    gsems = bufs_and_sems[NBUF + 1:2 * NBUF + 1]
    ssems = bufs_and_sems[2 * NBUF + 1:]
    c = lax.axis_index("c")
    s = lax.axis_index("s")
    wid = _worker(c, s)
    @pl.when(c == 0)
    def _():
        _each_tile_slab(s, lambda o, n: pltpu.sync_copy(
            y_hbm.at[pl.ds(o, n)], acc_sp.at[pl.ds(o, n)]))

    @pl.when(c != 0)
    def _():
        _each_tile_slab(s, lambda o, n: pltpu.sync_copy(
            zeros_hbm.at[pl.ds(o, n)], acc_sp.at[pl.ds(o, n)]))

    pltpu.sync_copy(ei_hbm.at[0, pl.ds(wid * NCH, NCH)], src_v)
    pltpu.sync_copy(ei_hbm.at[1, pl.ds(wid * NCH, NCH)], dst_v)
    plsc.subcore_barrier()

    for k in range(NBUF):
        pltpu.async_copy(y_hbm.at[src_v.at[k]], bufs[k], gsems[k])

    def body(j, carry):
        i = NBUF * j
        for k in range(NBUF):
            pltpu.make_async_copy(y_hbm.at[src_v.at[0]], bufs[k],
                                  gsems[k]).wait()
            pltpu.async_copy(bufs[k], acc_sp.at[dst_v.at[i + k]], ssems[k],
                             add=True)
        for k in range(NBUF):
            @pl.when(i + NBUF + k < NCH)
            def _(k=k):
                pltpu.make_async_copy(bufs[k], acc_sp.at[dst_v.at[0]],
                                      ssems[k]).wait()
                pltpu.async_copy(y_hbm.at[src_v.at[i + NBUF + k]], bufs[k],
                                 gsems[k])
        return carry

    lax.fori_loop(0, NCH // NBUF, body, 0)
    for k in range(NBUF):
        pltpu.make_async_copy(bufs[k], acc_sp.at[dst_v.at[0]],
                              ssems[k]).wait()
    plsc.subcore_barrier()
    _each_tile_slab(s, lambda o, n: pltpu.sync_copy(
        acc_sp.at[pl.ds(o, n)], out_hbm.at[c, pl.ds(o, n), pl.ds(0, d)]))


def _agg_kernel(y, ei3, zeros, d):
    return pl.kernel(
        functools.partial(_agg_body, d),
        out_type=jax.ShapeDtypeStruct((NC, N, 128), jnp.float32),
        mesh=_MESH,
        compiler_params=pltpu.CompilerParams(use_tc_tiling_on_sc=False),
        scratch_types=[
            pltpu.VMEM((NCH, CHUNK), jnp.int32),
            pltpu.VMEM((NCH, CHUNK), jnp.int32),
        ] + [pltpu.VMEM((CHUNK, d), jnp.float32) for _ in range(NBUF)] + [
            pltpu.VMEM_SHARED((N, d), jnp.float32),
        ] + [pltpu.SemaphoreType.DMA for _ in range(2 * NBUF)],
    )(y, ei3, zeros)


def _xw_body(x_ref, w1_ref, xw_ref):
    xw_ref[...] = lax.dot_general(x_ref[...], w1_ref[...],
                                  (((1,), (1,)), ((), ())),
                                  preferred_element_type=jnp.float32)


def _scale_body(xw_ref, degp_ref, y1_ref, dis_ref):
    deg = degp_ref[0, :, 0:1] + degp_ref[1, :, 0:1] + 1.0
    dis = lax.rsqrt(deg)
    y1_ref[...] = xw_ref[...] * dis
    dis_ref[...] = dis


def _tc2_body(aggp_ref, dis_ref, w2_ref, b1_ref, g1s_ref, be1_ref,
              y2_ref):
    dis = dis_ref[...]
    a = (aggp_ref[0, :, 0:64] + aggp_ref[1, :, 0:64]) * dis + b1_ref[...]
    h1 = jnp.maximum(a * g1s_ref[...] + be1_ref[...], 0.0)
    y2_ref[...] = lax.dot_general(h1, w2_ref[...], (((1,), (1,)), ((), ())),
                                  preferred_element_type=jnp.float32) * dis


def _tc3_body(aggp_ref, dis_ref, fcw_ref, b2_ref, g2s_ref, be2_ref,
              fcb_ref, out_ref):
    a = (aggp_ref[0, :, 0:32] + aggp_ref[1, :, 0:32]) * dis_ref[...] \
        + b2_ref[...]
    h2 = jnp.maximum(a * g2s_ref[...] + be2_ref[...], 0.0)
    out_ref[...] = jnp.sum(h2 * fcw_ref[...], axis=1, keepdims=True) \
        + fcb_ref[...]


def _tc_call(body, out_shapes):
    return pl.pallas_call(body, out_shape=out_shapes)


@jax.jit
def _run(x, edge_index, W1, b1, g1, be1, W2, b2, g2, be2, fcW, fcb):
    ei3 = edge_index.reshape(2, E // CHUNK, CHUNK)
    ones = jnp.ones((CHUNK, DEGW), jnp.float32)
    z8 = jnp.zeros((N, DEGW), jnp.float32)
    z64 = jnp.zeros((N, 64), jnp.float32)
    z32 = jnp.zeros((N, 32), jnp.float32)
    bn_s = 1.0 / jnp.sqrt(1.0 + BN_EPS)
    g1s = (g1 * bn_s).reshape(1, 64)
    g2s = (g2 * bn_s).reshape(1, 32)

    xw = _tc_call(_xw_body, jax.ShapeDtypeStruct((N, 64), jnp.float32))(x, W1)
    degp = _deg_kernel(ei3, ones, z8)

    y1, dis = _tc_call(_scale_body, (
        jax.ShapeDtypeStruct((N, 64), jnp.float32),
        jax.ShapeDtypeStruct((N, 1), jnp.float32),
    ))(xw, degp)

    agg1p = _agg_kernel(y1, ei3, z64, 64)

    y2 = _tc_call(_tc2_body, jax.ShapeDtypeStruct((N, 32), jnp.float32))(
        agg1p, dis, W2, b1.reshape(1, 64), g1s, be1.reshape(1, 64))

    agg2p = _agg_kernel(y2, ei3, z32, 32)

    out = _tc_call(_tc3_body, jax.ShapeDtypeStruct((N, 1), jnp.float32))(
        agg2p, dis, fcW, b2.reshape(1, 32), g2s, be2.reshape(1, 32),
        fcb.reshape(1, 1))
    return out


def kernel(x, edge_index, W1, b1, g1, be1, W2, b2, g2, be2, fcW, fcb):
    return _run(x, edge_index, W1, b1, g1, be1, W2, b2, g2, be2, fcW, fcb)


# speedup vs baseline: 51.7388x; 1.0401x over previous
import functools

import jax
import jax.numpy as jnp
from jax import lax
from jax.experimental import pallas as pl
from jax.experimental.pallas import tpu as pltpu
from jax.experimental.pallas import tpu_sc as plsc

N = 10000
E = 320000
BN_EPS = 1e-5

NC = 2
NS = 16
NW = NC * NS
EPW = E // NW
CHUNK = 125
NCH = EPW // CHUNK
RPT = 624
TAIL = N - NS * RPT
TAIL_OFF = NS * RPT
DEGW = 8
NBUF = 8

_MESH = plsc.VectorSubcoreMesh(core_axis_name="c", subcore_axis_name="s")


def _worker(core, sub):
    return sub * NC + core


def _each_tile_slab(s, fn):
    fn(pl.multiple_of(s * RPT, 8), RPT)

    @pl.when(s == NS - 1)
    def _():
        fn(TAIL_OFF, TAIL)


def _deg_body(ei_hbm, ones_hbm, zeros_hbm, out_hbm, dst_v, ones_v, deg_sp,
              sem):
    c = lax.axis_index("c")
    s = lax.axis_index("s")
    wid = _worker(c, s)
    _each_tile_slab(s, lambda o, n: pltpu.sync_copy(
        zeros_hbm.at[pl.ds(o, n)], deg_sp.at[pl.ds(o, n)]))
    pltpu.sync_copy(ones_hbm, ones_v)
    pltpu.sync_copy(ei_hbm.at[1, pl.ds(wid * NCH, NCH)], dst_v)
    plsc.subcore_barrier()

    def body(i, carry):
        pltpu.sync_copy(ones_v, deg_sp.at[dst_v.at[i]], add=True)
        return carry

    lax.fori_loop(0, NCH, body, 0)
    plsc.subcore_barrier()
    _each_tile_slab(s, lambda o, n: pltpu.sync_copy(
        deg_sp.at[pl.ds(o, n)], out_hbm.at[c, pl.ds(o, n)]))


def _deg_kernel(ei3, ones, zeros):
    return pl.kernel(
        _deg_body,
        out_type=jax.ShapeDtypeStruct((NC, N, DEGW), jnp.float32),
        mesh=_MESH,
        compiler_params=pltpu.CompilerParams(use_tc_tiling_on_sc=False),
        scratch_types=[
            pltpu.VMEM((NCH, CHUNK), jnp.int32),
            pltpu.VMEM((CHUNK, DEGW), jnp.float32),
            pltpu.VMEM_SHARED((N, DEGW), jnp.float32),
            pltpu.SemaphoreType.DMA,
        ],
    )(ei3, ones, zeros)


def _agg_body(d, y_hbm, ei_hbm, zeros_hbm, out_hbm, src_v, dst_v,
              *bufs_and_sems):
    bufs = bufs_and_sems[:NBUF]
    acc_sp = bufs_and_sems[NBUF]
    gsems = bufs_and_sems[NBUF + 1:2 * NBUF + 1]
    ssems = bufs_and_sems[2 * NBUF + 1:]
    c = lax.axis_index("c")
    s = lax.axis_index("s")
    wid = _worker(c, s)
    @pl.when(c == 0)
    def _():
        _each_tile_slab(s, lambda o, n: pltpu.sync_copy(
            y_hbm.at[pl.ds(o, n)], acc_sp.at[pl.ds(o, n)]))

    @pl.when(c != 0)
    def _():
        _each_tile_slab(s, lambda o, n: pltpu.sync_copy(
            zeros_hbm.at[pl.ds(o, n)], acc_sp.at[pl.ds(o, n)]))

    pltpu.sync_copy(ei_hbm.at[0, pl.ds(wid * NCH, NCH)], src_v)
    pltpu.sync_copy(ei_hbm.at[1, pl.ds(wid * NCH, NCH)], dst_v)
    plsc.subcore_barrier()

    for k in range(NBUF):
        pltpu.async_copy(y_hbm.at[src_v.at[k]], bufs[k], gsems[k])

    def body(j, carry):
        i = NBUF * j
        for k in range(NBUF):
            pltpu.make_async_copy(y_hbm.at[src_v.at[0]], bufs[k],
                                  gsems[k]).wait()
            pltpu.async_copy(bufs[k], acc_sp.at[dst_v.at[i + k]], ssems[k],
                             add=True)
        for k in range(NBUF):
            @pl.when(i + NBUF + k < NCH)
            def _(k=k):
                pltpu.make_async_copy(bufs[k], acc_sp.at[dst_v.at[0]],
                                      ssems[k]).wait()
                pltpu.async_copy(y_hbm.at[src_v.at[i + NBUF + k]], bufs[k],
                                 gsems[k])
        return carry

    lax.fori_loop(0, NCH // NBUF, body, 0)
    for k in range(NBUF):
        pltpu.make_async_copy(bufs[k], acc_sp.at[dst_v.at[0]],
                              ssems[k]).wait()
    plsc.subcore_barrier()
    _each_tile_slab(s, lambda o, n: pltpu.sync_copy(
        acc_sp.at[pl.ds(o, n)], out_hbm.at[c, pl.ds(o, n), pl.ds(0, d)]))


def _agg_kernel(y, ei3, zeros, d):
    return pl.kernel(
        functools.partial(_agg_body, d),
        out_type=jax.ShapeDtypeStruct((NC, N, 128), jnp.float32),
        mesh=_MESH,
        compiler_params=pltpu.CompilerParams(use_tc_tiling_on_sc=False),
        scratch_types=[
            pltpu.VMEM((NCH, CHUNK), jnp.int32),
            pltpu.VMEM((NCH, CHUNK), jnp.int32),
        ] + [pltpu.VMEM((CHUNK, d), jnp.float32) for _ in range(NBUF)] + [
            pltpu.VMEM_SHARED((N, d), jnp.float32),
        ] + [pltpu.SemaphoreType.DMA for _ in range(2 * NBUF)],
    )(y, ei3, zeros)


def _xw_body(x_ref, w1_ref, xw_ref):
    xw_ref[...] = lax.dot_general(x_ref[...], w1_ref[...],
                                  (((1,), (1,)), ((), ())),
                                  preferred_element_type=jnp.float32)


def _scale_body(xw_ref, degp_ref, y1_ref, dis_ref):
    deg = degp_ref[0, :, 0:1] + degp_ref[1, :, 0:1] + 1.0
    dis = lax.rsqrt(deg)
    y1_ref[...] = xw_ref[...] * dis
    dis_ref[...] = dis


def _tc2_body(aggp_ref, dis_ref, w2_ref, b1_ref, g1s_ref, be1_ref,
              y2_ref):
    dis = dis_ref[...]
    a = (aggp_ref[0, :, 0:64] + aggp_ref[1, :, 0:64]) * dis + b1_ref[...]
    h1 = jnp.maximum(a * g1s_ref[...] + be1_ref[...], 0.0)
    y2_ref[...] = lax.dot_general(h1, w2_ref[...], (((1,), (1,)), ((), ())),
                                  preferred_element_type=jnp.float32) * dis


def _tc3_body(aggp_ref, dis_ref, fcw_ref, b2_ref, g2s_ref, be2_ref,
              fcb_ref, out_ref):
    a = (aggp_ref[0, :, 0:32] + aggp_ref[1, :, 0:32]) * dis_ref[...] \
        + b2_ref[...]
    h2 = jnp.maximum(a * g2s_ref[...] + be2_ref[...], 0.0)
    out_ref[...] = jnp.sum(h2 * fcw_ref[...], axis=1, keepdims=True) \
        + fcb_ref[...]


def _tc_call(body, out_shapes):
    return pl.pallas_call(body, out_shape=out_shapes)


@jax.jit
def _run(x, edge_index, W1, b1, g1, be1, W2, b2, g2, be2, fcW, fcb):
    ei3 = edge_index.reshape(2, E // CHUNK, CHUNK)
    ones = jnp.ones((CHUNK, DEGW), jnp.float32)
    z8 = jnp.zeros((N, DEGW), jnp.float32)
    z64 = jnp.zeros((N, 64), jnp.float32)
    z32 = jnp.zeros((N, 32), jnp.float32)
    bn_s = 1.0 / jnp.sqrt(1.0 + BN_EPS)
    g1s = (g1 * bn_s).reshape(1, 64)
    g2s = (g2 * bn_s).reshape(1, 32)

    xw = _tc_call(_xw_body, jax.ShapeDtypeStruct((N, 64), jnp.float32))(x, W1)
    degp = _deg_kernel(ei3, ones, z8)

    y1, dis = _tc_call(_scale_body, (
        jax.ShapeDtypeStruct((N, 64), jnp.float32),
        jax.ShapeDtypeStruct((N, 1), jnp.float32),
    ))(xw, degp)

    agg1p = _agg_kernel(y1, ei3, z64, 64)

    y2 = _tc_call(_tc2_body, jax.ShapeDtypeStruct((N, 32), jnp.float32))(
        agg1p, dis, W2, b1.reshape(1, 64), g1s, be1.reshape(1, 64))

    agg2p = _agg_kernel(y2, ei3, z32, 32)

    out = _tc_call(_tc3_body, jax.ShapeDtypeStruct((N, 1), jnp.float32))(
        agg2p, dis, fcW, b2.reshape(1, 32), g2s, be2.reshape(1, 32),
        fcb.reshape(1, 1))
    return out


def kernel(x, edge_index, W1, b1, g1, be1, W2, b2, g2, be2, fcW, fcb):
    return _run(x, edge_index, W1, b1, g1, be1, W2, b2, g2, be2, fcW, fcb)
